# baseline (speedup 1.0000x reference)
import jax
import jax.numpy as jnp
from jax import lax
from jax.experimental import pallas as pl
from jax.experimental.pallas import tpu as pltpu
from jax.experimental.pallas import tpu_sc as plsc

N = 50000
NE = 800000
D_IN = 64
D_E = 16
H = 128
L = 4
D_OUT = 32

NBLK = 1000
EBLK = 8000

NT = 16
NR = 6
NQ = 8448
NP = NR * NQ
NEP = 819200
CHUNK = 128
EPT = NEP // NT
NCH = EPT // CHUNK
ACCR = NQ + 128
ZROW = ACCR // NT
WROW = NQ // NT
DUMMY = N


def _ln(x, g, b, eps=1e-5):
    m = x.mean(-1, keepdims=True)
    v = ((x - m) ** 2).mean(-1, keepdims=True)
    return (x - m) * lax.rsqrt(v + eps) * g + b


def _enc_body(x_ref, w1_ref, b1_ref, g_ref, be_ref, w2_ref, b2_ref, o_ref):
    x = x_ref[...]
    h = jnp.maximum(x @ w1_ref[...] + b1_ref[...], 0.0)
    h = _ln(h, g_ref[...], be_ref[...])
    o_ref[...] = jnp.maximum(h @ w2_ref[...] + b2_ref[...], 0.0)


def _encoder(x, w1, b1, g, be, w2, b2):
    grid = (N // NBLK,)
    full = lambda r, c: pl.BlockSpec((r, c), lambda i: (0, 0))
    return pl.pallas_call(
        _enc_body,
        grid=grid,
        in_specs=[
            pl.BlockSpec((NBLK, D_IN), lambda i: (i, 0)),
            full(D_IN, H), full(1, H), full(1, H), full(1, H),
            full(H, H), full(1, H),
        ],
        out_specs=pl.BlockSpec((NBLK, H), lambda i: (i, 0)),
        out_shape=jax.ShapeDtypeStruct((N, H), jnp.float32),
    )(x, w1, b1.reshape(1, H), g.reshape(1, H), be.reshape(1, H),
      w2, b2.reshape(1, H))


def _ab_body(h_ref, w_ref, b_ref, a_ref, bo_ref):
    h = h_ref[...]
    ab = h @ w_ref[...] + b_ref[...]
    a_ref[...] = ab[:, :H]
    bo_ref[...] = ab[:, H:]


def _ab(h, w_ab, bias_ab):
    grid = (N // NBLK,)
    return pl.pallas_call(
        _ab_body,
        grid=grid,
        in_specs=[
            pl.BlockSpec((NBLK, H), lambda i: (i, 0)),
            pl.BlockSpec((H, 2 * H), lambda i: (0, 0)),
            pl.BlockSpec((1, 2 * H), lambda i: (0, 0)),
        ],
        out_specs=[
            pl.BlockSpec((NBLK, H), lambda i: (i, 0)),
            pl.BlockSpec((NBLK, H), lambda i: (i, 0)),
        ],
        out_shape=[
            jax.ShapeDtypeStruct((NP, H), jnp.float32),
            jax.ShapeDtypeStruct((NP, H), jnp.float32),
        ],
    )(h, w_ab, bias_ab)


def _e_body(e_ref, w_ref, o_ref):
    o_ref[...] = e_ref[...] @ w_ref[...]


def _eproj(ef, w_e):
    grid = (NE // EBLK,)
    return pl.pallas_call(
        _e_body,
        grid=grid,
        in_specs=[
            pl.BlockSpec((EBLK, D_E), lambda i: (i, 0)),
            pl.BlockSpec((D_E, H), lambda i: (0, 0)),
        ],
        out_specs=pl.BlockSpec((EBLK, H), lambda i: (i, 0)),
        out_shape=jax.ShapeDtypeStruct((NEP, H), jnp.float32),
    )(ef, w_e)


def _upd_body(h_ref, p_ref, cnt_ref, w2_ref, b2_ref, wuh_ref, wua_ref,
              ub_ref, g_ref, b_ref, o_ref):
    cnt = cnt_ref[...][:, 0:1]
    agg = (p_ref[...] @ w2_ref[...] + cnt * b2_ref[...]) / (cnt + 1e-8)
    h = h_ref[...]
    hn = jnp.maximum(h @ wuh_ref[...] + agg @ wua_ref[...] + ub_ref[...], 0.0)
    o_ref[...] = _ln(h + hn, g_ref[...], b_ref[...])


def _update(h, p, cnt, w2, b2, wuh, wua, ub, g, b):
    grid = (N // NBLK,)
    full = lambda r, c: pl.BlockSpec((r, c), lambda i: (0, 0))
    return pl.pallas_call(
        _upd_body,
        grid=grid,
        in_specs=[
            pl.BlockSpec((NBLK, H), lambda i: (i, 0)),
            pl.BlockSpec((NBLK, H), lambda i: (i, 0)),
            pl.BlockSpec((NBLK, H), lambda i: (i, 0)),
            full(H, H), full(1, H), full(H, H), full(H, H),
            full(1, H), full(1, H), full(1, H),
        ],
        out_specs=pl.BlockSpec((NBLK, H), lambda i: (i, 0)),
        out_shape=jax.ShapeDtypeStruct((N, H), jnp.float32),
    )(h, p, cnt, w2, b2.reshape(1, H), wuh, wua,
      ub.reshape(1, H), g.reshape(1, H), b.reshape(1, H))


def _pool_body(h_ref, aw1_ref, ab1_ref, aw2_ref, ab2_ref,
               ow1_ref, ob1_ref, ow2_ref, ob2_ref, o_ref,
               sexp_ref, gacc_ref):
    i = pl.program_id(0)

    @pl.when(i == 0)
    def _():
        sexp_ref[0, 0] = 0.0
        gacc_ref[...] = jnp.zeros_like(gacc_ref)

    h = h_ref[...]
    t = jnp.tanh(h @ aw1_ref[...] + ab1_ref[...])
    att = t @ aw2_ref[...] + ab2_ref[...]
    w = jnp.exp(att)
    sexp_ref[0, 0] += jnp.sum(w)
    gacc_ref[...] += (w * h).sum(axis=0, keepdims=True)

    @pl.when(i == pl.num_programs(0) - 1)
    def _():
        g = gacc_ref[...] / sexp_ref[0, 0]
        o_ref[...] = jnp.maximum(g @ ow1_ref[...] + ob1_ref[...], 0.0) \
            @ ow2_ref[...] + ob2_ref[...]


def _pool_head(h, aw1, ab1, aw2, ab2, ow1, ob1, ow2, ob2):
    grid = (N // NBLK,)
    full = lambda r, c: pl.BlockSpec((r, c), lambda i: (0, 0))
    return pl.pallas_call(
        _pool_body,
        grid=grid,
        in_specs=[
            pl.BlockSpec((NBLK, H), lambda i: (i, 0)),
            full(H, H // 2), full(1, H // 2), full(H // 2, 1), full(1, 1),
            full(H, H), full(1, H), full(H, D_OUT), full(1, D_OUT),
        ],
        out_specs=pl.BlockSpec((1, D_OUT), lambda i: (0, 0)),
        out_shape=jax.ShapeDtypeStruct((1, D_OUT), jnp.float32),
        scratch_shapes=[
            pltpu.SMEM((1, 1), jnp.float32),
            pltpu.VMEM((1, H), jnp.float32),
        ],
    )(h, aw1, ab1.reshape(1, H // 2), aw2, ab2.reshape(1, 1),
      ow1, ob1.reshape(1, H), ow2, ob2.reshape(1, D_OUT))


_SC_MESH = plsc.VectorSubcoreMesh(core_axis_name="c", subcore_axis_name="s")


def _zero_acc_rows(zbuf, acc, t):
    for z in range(ZROW // CHUNK):
        pltpu.sync_copy(zbuf, acc.at[pl.ds(ZROW * t + CHUNK * z, CHUNK)])
    rem = ZROW % CHUNK
    if rem:
        pltpu.sync_copy(zbuf.at[pl.ds(0, rem)],
                        acc.at[pl.ds(ZROW * t + ZROW - rem, rem)])


def _range_mask(d_idx, d_eff, lo):
    for v in range(8):
        sl = pl.ds(16 * v, 16)
        d = d_idx[0, sl]
        inr = (d >= lo) & (d < lo + NQ)
        d_eff[0, sl] = jnp.where(inr, d - lo, NQ)


def _cnt_body(dstR, cnt, ones, zbuf, d_idx, d_eff, acc):
    c = lax.axis_index("c")
    t = lax.axis_index("s")

    def fill(k, _):
        for v in range(8):
            ones[k, pl.ds(16 * v, 16)] = jnp.ones((16,), jnp.float32)
            zbuf[k, pl.ds(16 * v, 16)] = jnp.zeros((16,), jnp.float32)
        return 0
    lax.fori_loop(0, CHUNK, fill, 0)

    for p in range(NR // 2):
        q = (NR // 2) * c + p
        lo = q * NQ
        _zero_acc_rows(zbuf, acc, t)
        plsc.subcore_barrier()

        def chunk(i, _):
            rows = t * (EPT // CHUNK) + i
            pltpu.sync_copy(dstR.at[pl.ds(rows, 1)], d_idx)
            _range_mask(d_idx, d_eff, lo)
            pltpu.sync_copy(ones, acc.at[d_eff.at[0]], add=True)
            return 0
        lax.fori_loop(0, NCH, chunk, 0)

        plsc.subcore_barrier()
        pltpu.sync_copy(acc.at[pl.ds(WROW * t, WROW)],
                        cnt.at[pl.ds(lo + WROW * t, WROW)])
        plsc.subcore_barrier()


def _counts(dstR):
    return pl.kernel(
        _cnt_body,
        out_type=jax.ShapeDtypeStruct((NP, H), jnp.float32),
        mesh=_SC_MESH,
        scratch_types=[
            pltpu.VMEM((CHUNK, H), jnp.float32),
            pltpu.VMEM((CHUNK, H), jnp.float32),
            pltpu.VMEM((1, CHUNK), jnp.int32),
            pltpu.VMEM((1, CHUNK), jnp.int32),
            pltpu.VMEM_SHARED((ACCR, H), jnp.float32),
        ],
    )(dstR)


def _edge_body(at, bt, et, srcR, dstR, p_out,
               a_buf, b_buf, e_buf, s_idx, d_idx, d_eff, acc, sem):
    c = lax.axis_index("c")
    t = lax.axis_index("s")

    for p in range(NR // 2):
        q = (NR // 2) * c + p
        lo = q * NQ

        def zfill(k, _):
            for v in range(8):
                e_buf[k, pl.ds(16 * v, 16)] = jnp.zeros((16,), jnp.float32)
            return 0
        lax.fori_loop(0, CHUNK, zfill, 0)
        _zero_acc_rows(e_buf, acc, t)
        plsc.subcore_barrier()

        def chunk(i, _):
            rows = t * (EPT // CHUNK) + i
            pltpu.sync_copy(srcR.at[pl.ds(rows, 1)], s_idx)
            pltpu.sync_copy(dstR.at[pl.ds(rows, 1)], d_idx)
            _range_mask(d_idx, d_eff, lo)

            cps = [
                pltpu.async_copy(at.at[s_idx.at[0]], a_buf, sem),
                pltpu.async_copy(bt.at[d_idx.at[0]], b_buf, sem),
                pltpu.async_copy(
                    et.at[pl.ds(t * EPT + i * CHUNK, CHUNK)], e_buf, sem),
            ]
            for cp in cps:
                cp.wait()

            def rbody(k, _):
                for v in range(8):
                    sl = pl.ds(16 * v, 16)
                    s = a_buf[k, sl] + b_buf[k, sl] + e_buf[k, sl]
                    a_buf[k, sl] = jnp.maximum(s, 0.0)
                return 0
            lax.fori_loop(0, CHUNK, rbody, 0)

            pltpu.sync_copy(a_buf, acc.at[d_eff.at[0]], add=True)
            return 0
        lax.fori_loop(0, NCH, chunk, 0)

        plsc.subcore_barrier()
        pltpu.sync_copy(acc.at[pl.ds(WROW * t, WROW)],
                        p_out.at[pl.ds(lo + WROW * t, WROW)])
        plsc.subcore_barrier()


def _sc_edges(at, bt, et, srcR, dstR):
    return pl.kernel(
        _edge_body,
        out_type=jax.ShapeDtypeStruct((NP, H), jnp.float32),
        mesh=_SC_MESH,
        scratch_types=[
            pltpu.VMEM((CHUNK, H), jnp.float32),
            pltpu.VMEM((CHUNK, H), jnp.float32),
            pltpu.VMEM((CHUNK, H), jnp.float32),
            pltpu.VMEM((1, CHUNK), jnp.int32),
            pltpu.VMEM((1, CHUNK), jnp.int32),
            pltpu.VMEM((1, CHUNK), jnp.int32),
            pltpu.VMEM_SHARED((ACCR, H), jnp.float32),
            pltpu.SemaphoreType.DMA,
        ],
    )(at, bt, et, srcR, dstR)


def kernel(node_features, edge_index, edge_features, constraint_types,
           enc_W1, enc_b1, enc_g, enc_be, enc_W2, enc_b2,
           msg_W1, msg_b1, msg_W2, msg_b2, upd_W, upd_b, ln_g, ln_b,
           att_W1, att_b1, att_W2, att_b2,
           out_W1, out_b1, out_W2, out_b2):
    src = edge_index[0]
    dst = edge_index[1]
    pad = jnp.full((NEP - NE,), DUMMY, jnp.int32)
    srcR = jnp.concatenate([src, pad]).reshape(NEP // CHUNK, CHUNK)
    dstR = jnp.concatenate([dst, pad]).reshape(NEP // CHUNK, CHUNK)

    h = _encoder(node_features, enc_W1, enc_b1, enc_g, enc_be, enc_W2, enc_b2)
    cnt = _counts(dstR)

    for i in range(L):
        w1 = msg_W1[i]
        w_ab = jnp.concatenate([w1[:H], w1[H:2 * H]], axis=1)
        bias_ab = jnp.concatenate(
            [msg_b1[i], jnp.zeros_like(msg_b1[i])]).reshape(1, 2 * H)
        a, b = _ab(h, w_ab, bias_ab)
        e = _eproj(edge_features, w1[2 * H:])
        p = _sc_edges(a, b, e, srcR, dstR)
        h = _update(h, p, cnt, msg_W2[i], msg_b2[i],
                    upd_W[i][:H], upd_W[i][H:], upd_b[i], ln_g[i], ln_b[i])

    out = _pool_head(h, att_W1, att_b1, att_W2, att_b2,
                     out_W1, out_b1, out_W2, out_b2)
    return out.reshape(D_OUT)


# speedup vs baseline: 2.0728x; 2.0728x over previous
import jax
import jax.numpy as jnp
from jax import lax
from jax.experimental import pallas as pl
from jax.experimental.pallas import tpu as pltpu
from jax.experimental.pallas import tpu_sc as plsc

N = 50000
NE = 800000
D_IN = 64
D_E = 16
H = 128
L = 4
D_OUT = 32

NBLK = 1000
EBLK = 8000

NT = 16
NR = 16
NQ = 3200
NP = NR * NQ
NEP = 819200
EPT = NEP // NT
NJR = EPT // 128
ACCR = NQ + 128
ZROW = ACCR // NT
WROW = NQ // NT
DUMMY = N


def _ln(x, g, b, eps=1e-5):
    m = x.mean(-1, keepdims=True)
    v = ((x - m) ** 2).mean(-1, keepdims=True)
    return (x - m) * lax.rsqrt(v + eps) * g + b


def _enc_body(x_ref, w1_ref, b1_ref, g_ref, be_ref, w2_ref, b2_ref, o_ref):
    x = x_ref[...]
    h = jnp.maximum(x @ w1_ref[...] + b1_ref[...], 0.0)
    h = _ln(h, g_ref[...], be_ref[...])
    o_ref[...] = jnp.maximum(h @ w2_ref[...] + b2_ref[...], 0.0)


def _encoder(x, w1, b1, g, be, w2, b2):
    grid = (N // NBLK,)
    full = lambda r, c: pl.BlockSpec((r, c), lambda i: (0, 0))
    return pl.pallas_call(
        _enc_body,
        grid=grid,
        in_specs=[
            pl.BlockSpec((NBLK, D_IN), lambda i: (i, 0)),
            full(D_IN, H), full(1, H), full(1, H), full(1, H),
            full(H, H), full(1, H),
        ],
        out_specs=pl.BlockSpec((NBLK, H), lambda i: (i, 0)),
        out_shape=jax.ShapeDtypeStruct((N, H), jnp.float32),
    )(x, w1, b1.reshape(1, H), g.reshape(1, H), be.reshape(1, H),
      w2, b2.reshape(1, H))


def _ab_body(h_ref, w_ref, b_ref, a_ref, bo_ref):
    h = h_ref[...]
    ab = h @ w_ref[...] + b_ref[...]
    a_ref[...] = ab[:, :H]
    bo_ref[...] = ab[:, H:]


def _ab(h, w_ab, bias_ab):
    grid = (N // NBLK,)
    return pl.pallas_call(
        _ab_body,
        grid=grid,
        in_specs=[
            pl.BlockSpec((NBLK, H), lambda i: (i, 0)),
            pl.BlockSpec((H, 2 * H), lambda i: (0, 0)),
            pl.BlockSpec((1, 2 * H), lambda i: (0, 0)),
        ],
        out_specs=[
            pl.BlockSpec((NBLK, H), lambda i: (i, 0)),
            pl.BlockSpec((NBLK, H), lambda i: (i, 0)),
        ],
        out_shape=[
            jax.ShapeDtypeStruct((NP, H), jnp.float32),
            jax.ShapeDtypeStruct((NP, H), jnp.float32),
        ],
    )(h, w_ab, bias_ab)


def _e_body(e_ref, w_ref, o_ref):
    o_ref[...] = e_ref[...] @ w_ref[...]


def _eproj(ef, w_e):
    grid = (NE // EBLK,)
    return pl.pallas_call(
        _e_body,
        grid=grid,
        in_specs=[
            pl.BlockSpec((EBLK, D_E), lambda i: (i, 0)),
            pl.BlockSpec((D_E, H), lambda i: (0, 0)),
        ],
        out_specs=pl.BlockSpec((EBLK, H), lambda i: (i, 0)),
        out_shape=jax.ShapeDtypeStruct((NEP, H), jnp.float32),
    )(ef, w_e)


def _upd_body(h_ref, p_ref, cnt_ref, w2_ref, b2_ref, wuh_ref, wua_ref,
              ub_ref, g_ref, b_ref, o_ref):
    cnt = cnt_ref[...][:, 0:1]
    agg = (p_ref[...] @ w2_ref[...] + cnt * b2_ref[...]) / (cnt + 1e-8)
    h = h_ref[...]
    hn = jnp.maximum(h @ wuh_ref[...] + agg @ wua_ref[...] + ub_ref[...], 0.0)
    o_ref[...] = _ln(h + hn, g_ref[...], b_ref[...])


def _update(h, p, cnt, w2, b2, wuh, wua, ub, g, b):
    grid = (N // NBLK,)
    full = lambda r, c: pl.BlockSpec((r, c), lambda i: (0, 0))
    return pl.pallas_call(
        _upd_body,
        grid=grid,
        in_specs=[
            pl.BlockSpec((NBLK, H), lambda i: (i, 0)),
            pl.BlockSpec((NBLK, H), lambda i: (i, 0)),
            pl.BlockSpec((NBLK, H), lambda i: (i, 0)),
            full(H, H), full(1, H), full(H, H), full(H, H),
            full(1, H), full(1, H), full(1, H),
        ],
        out_specs=pl.BlockSpec((NBLK, H), lambda i: (i, 0)),
        out_shape=jax.ShapeDtypeStruct((N, H), jnp.float32),
    )(h, p, cnt, w2, b2.reshape(1, H), wuh, wua,
      ub.reshape(1, H), g.reshape(1, H), b.reshape(1, H))


def _pool_body(h_ref, aw1_ref, ab1_ref, aw2_ref, ab2_ref,
               ow1_ref, ob1_ref, ow2_ref, ob2_ref, o_ref,
               sexp_ref, gacc_ref):
    i = pl.program_id(0)

    @pl.when(i == 0)
    def _():
        sexp_ref[0, 0] = 0.0
        gacc_ref[...] = jnp.zeros_like(gacc_ref)

    h = h_ref[...]
    t = jnp.tanh(h @ aw1_ref[...] + ab1_ref[...])
    att = t @ aw2_ref[...] + ab2_ref[...]
    w = jnp.exp(att)
    sexp_ref[0, 0] += jnp.sum(w)
    gacc_ref[...] += (w * h).sum(axis=0, keepdims=True)

    @pl.when(i == pl.num_programs(0) - 1)
    def _():
        g = gacc_ref[...] / sexp_ref[0, 0]
        o_ref[...] = jnp.maximum(g @ ow1_ref[...] + ob1_ref[...], 0.0) \
            @ ow2_ref[...] + ob2_ref[...]


def _pool_head(h, aw1, ab1, aw2, ab2, ow1, ob1, ow2, ob2):
    grid = (N // NBLK,)
    full = lambda r, c: pl.BlockSpec((r, c), lambda i: (0, 0))
    return pl.pallas_call(
        _pool_body,
        grid=grid,
        in_specs=[
            pl.BlockSpec((NBLK, H), lambda i: (i, 0)),
            full(H, H // 2), full(1, H // 2), full(H // 2, 1), full(1, 1),
            full(H, H), full(1, H), full(H, D_OUT), full(1, D_OUT),
        ],
        out_specs=pl.BlockSpec((1, D_OUT), lambda i: (0, 0)),
        out_shape=jax.ShapeDtypeStruct((1, D_OUT), jnp.float32),
        scratch_shapes=[
            pltpu.SMEM((1, 1), jnp.float32),
            pltpu.VMEM((1, H), jnp.float32),
        ],
    )(h, aw1, ab1.reshape(1, H // 2), aw2, ab2.reshape(1, 1),
      ow1, ob1.reshape(1, H), ow2, ob2.reshape(1, D_OUT))


_SC_MESH = plsc.VectorSubcoreMesh(core_axis_name="c", subcore_axis_name="s")


def _dyn_gather(x, idx):
    dnums = lax.GatherDimensionNumbers(
        offset_dims=(), collapsed_slice_dims=(0,), start_index_map=(0,))
    return lax.gather(x, idx[:, None], dnums, (1,),
                      mode=lax.GatherScatterMode.PROMISE_IN_BOUNDS)


def _prefix16(mi, lane):
    cs = mi
    for sh in (1, 2, 4, 8):
        g = _dyn_gather(cs, jnp.maximum(lane - sh, 0))
        cs = cs + jnp.where(lane >= sh, g, 0)
    return cs


def _compact_perm(cs, lane):
    lo = jnp.zeros((16,), jnp.int32)
    tgt = lane + 1
    for step in (8, 4, 2, 1):
        probe = jnp.minimum(lo + (step - 1), 15)
        val = _dyn_gather(cs, probe)
        lo = jnp.where(val < tgt, lo + step, lo)
    return jnp.minimum(lo, 15)


def _vzero(buf, rows):
    def body(k, _):
        for v in range(8):
            buf[k, pl.ds(16 * v, 16)] = jnp.zeros((16,), jnp.float32)
        return 0
    lax.fori_loop(0, rows, body, 0)


def _zero_acc_rows(zbuf, acc, t):
    pltpu.sync_copy(zbuf, acc.at[pl.ds(ZROW * t, 128)])
    pltpu.sync_copy(zbuf.at[pl.ds(0, ZROW - 128)],
                    acc.at[pl.ds(ZROW * t + 128, ZROW - 128)])


def _stage_full(fsrc, fdstl, feid, i2s, i2dl, i2dg, i2e, lo):
    for v in range(8):
        sl = pl.ds(16 * v, 16)
        i2s[0, sl] = fsrc[sl]
        dl = fdstl[sl]
        i2dl[0, sl] = dl
        i2dg[0, sl] = dl + lo
        i2e[0, sl] = feid[sl]
    for v in range(8):
        fsrc[pl.ds(16 * v, 16)] = fsrc[pl.ds(128 + 16 * v, 16)]
        fdstl[pl.ds(16 * v, 16)] = fdstl[pl.ds(128 + 16 * v, 16)]
        feid[pl.ds(16 * v, 16)] = feid[pl.ds(128 + 16 * v, 16)]


def _stage_tail(fsrc, fdstl, feid, i2s, i2dl, i2dg, i2e, lo, cur):
    for v in range(8):
        sl = pl.ds(16 * v, 16)
        w = lax.iota(jnp.int32, 16) + (16 * v) < cur
        i2s[0, sl] = jnp.where(w, fsrc[sl], DUMMY)
        dl = fdstl[sl]
        i2dl[0, sl] = jnp.where(w, dl, NQ)
        i2dg[0, sl] = jnp.where(w, dl + lo, DUMMY)
        i2e[0, sl] = jnp.where(w, feid[sl], NE)


def _fire_batch(at, bt, et, i2s, i2dl, i2dg, i2e, a_buf, b_buf, e_buf,
                acc, sem):
    cps = [
        pltpu.async_copy(at.at[i2s.at[0]], a_buf, sem),
        pltpu.async_copy(bt.at[i2dg.at[0]], b_buf, sem),
        pltpu.async_copy(et.at[i2e.at[0]], e_buf, sem),
    ]
    for cp in cps:
        cp.wait()

    def rbody(k, _):
        for v in range(8):
            sl = pl.ds(16 * v, 16)
            s = a_buf[k, sl] + b_buf[k, sl] + e_buf[k, sl]
            a_buf[k, sl] = jnp.maximum(s, 0.0)
        return 0
    lax.fori_loop(0, 128, rbody, 0)

    pltpu.sync_copy(a_buf, acc.at[i2dl.at[0]], add=True)


def _edge_body(at, bt, et, srcR, dstR, p_out,
               a_buf, b_buf, e_buf, sbuf, dbuf, fsrc, fdstl, feid,
               i2s, i2dl, i2dg, i2e, xbuf, acc, sem):
    c = lax.axis_index("c")
    t = lax.axis_index("s")

    def one_pass(p, _):
        q = (NR // 2) * c + p
        lo = q * NQ

        _vzero(e_buf, 128)
        _zero_acc_rows(e_buf, acc, t)
        plsc.subcore_barrier()

        def jrow(j, cur):
            @pl.when(j % 8 == 0)
            def _():
                row0 = pl.multiple_of(t * NJR + j, 8)
                pltpu.sync_copy(srcR.at[pl.ds(row0, 8)], sbuf)
                pltpu.sync_copy(dstR.at[pl.ds(row0, 8)], dbuf)
            jj = j % 8
            base = t * EPT + j * 128
            lane = lax.iota(jnp.int32, 16)
            for v in range(8):
                sl = pl.ds(16 * v, 16)
                d = dbuf[jj, sl]
                m = (d >= lo) & (d < lo + NQ)
                cs = _prefix16(jnp.where(m, 1, 0), lane)
                xbuf[pl.ds(0, 16)] = cs
                perm = _compact_perm(cs, lane)
                fsrc[pl.ds(cur, 16)] = _dyn_gather(sbuf[jj, sl], perm)
                fdstl[pl.ds(cur, 16)] = _dyn_gather(d - lo, perm)
                fe = perm + (base + 16 * v)
                feid[pl.ds(cur, 16)] = fe
                cur = cur + xbuf[pl.ds(0, 16)][15]

            full = cur >= 128

            @pl.when(full)
            def _():
                _stage_full(fsrc, fdstl, feid, i2s, i2dl, i2dg, i2e, lo)
                _fire_batch(at, bt, et, i2s, i2dl, i2dg, i2e,
                            a_buf, b_buf, e_buf, acc, sem)
            return jnp.where(full, cur - 128, cur)

        cur = lax.fori_loop(0, NJR, jrow, jnp.int32(0))

        @pl.when(cur > 0)
        def _():
            _stage_tail(fsrc, fdstl, feid, i2s, i2dl, i2dg, i2e, lo, cur)
            _fire_batch(at, bt, et, i2s, i2dl, i2dg, i2e,
                        a_buf, b_buf, e_buf, acc, sem)

        plsc.subcore_barrier()
        pltpu.sync_copy(acc.at[pl.ds(WROW * t, WROW)],
                        p_out.at[pl.ds(pl.multiple_of(lo + WROW * t, 8),
                                       WROW)])
        plsc.subcore_barrier()
        return 0

    lax.fori_loop(0, NR // 2, one_pass, 0)


def _sc_edges(at, bt, et, srcR, dstR):
    return pl.kernel(
        _edge_body,
        out_type=jax.ShapeDtypeStruct((NP, H), jnp.float32),
        mesh=_SC_MESH,
        scratch_types=[
            pltpu.VMEM((128, H), jnp.float32),
            pltpu.VMEM((128, H), jnp.float32),
            pltpu.VMEM((128, H), jnp.float32),
            pltpu.VMEM((8, 128), jnp.int32),
            pltpu.VMEM((8, 128), jnp.int32),
            pltpu.VMEM((272,), jnp.int32),
            pltpu.VMEM((272,), jnp.int32),
            pltpu.VMEM((272,), jnp.int32),
            pltpu.VMEM((1, 128), jnp.int32),
            pltpu.VMEM((1, 128), jnp.int32),
            pltpu.VMEM((1, 128), jnp.int32),
            pltpu.VMEM((1, 128), jnp.int32),
            pltpu.VMEM((16,), jnp.int32),
            pltpu.VMEM_SHARED((ACCR, H), jnp.float32),
            pltpu.SemaphoreType.DMA,
        ],
    )(at, bt, et, srcR, dstR)


def _cnt_body(dstR, cnt, ones, zbuf, dbuf, fdstl, i2dl, xbuf, acc):
    c = lax.axis_index("c")
    t = lax.axis_index("s")

    def ofill(k, _):
        for v in range(8):
            ones[k, pl.ds(16 * v, 16)] = jnp.ones((16,), jnp.float32)
        return 0
    lax.fori_loop(0, 128, ofill, 0)
    _vzero(zbuf, 128)

    def one_pass(p, _):
        q = (NR // 2) * c + p
        lo = q * NQ

        _zero_acc_rows(zbuf, acc, t)
        plsc.subcore_barrier()

        def jrow(j, cur):
            @pl.when(j % 8 == 0)
            def _():
                pltpu.sync_copy(
                    dstR.at[pl.ds(pl.multiple_of(t * NJR + j, 8), 8)], dbuf)
            jj = j % 8
            lane = lax.iota(jnp.int32, 16)
            for v in range(8):
                d = dbuf[jj, pl.ds(16 * v, 16)]
                m = (d >= lo) & (d < lo + NQ)
                cs = _prefix16(jnp.where(m, 1, 0), lane)
                xbuf[pl.ds(0, 16)] = cs
                perm = _compact_perm(cs, lane)
                fdstl[pl.ds(cur, 16)] = _dyn_gather(d - lo, perm)
                cur = cur + xbuf[pl.ds(0, 16)][15]

            full = cur >= 128

            @pl.when(full)
            def _():
                for v in range(8):
                    i2dl[0, pl.ds(16 * v, 16)] = fdstl[pl.ds(16 * v, 16)]
                for v in range(8):
                    fdstl[pl.ds(16 * v, 16)] = fdstl[pl.ds(128 + 16 * v, 16)]
                pltpu.sync_copy(ones, acc.at[i2dl.at[0]], add=True)
            return jnp.where(full, cur - 128, cur)

        cur = lax.fori_loop(0, NJR, jrow, jnp.int32(0))

        @pl.when(cur > 0)
        def _():
            for v in range(8):
                sl = pl.ds(16 * v, 16)
                w = lax.iota(jnp.int32, 16) + (16 * v) < cur
                i2dl[0, sl] = jnp.where(w, fdstl[sl], NQ)
            pltpu.sync_copy(ones, acc.at[i2dl.at[0]], add=True)

        plsc.subcore_barrier()
        pltpu.sync_copy(acc.at[pl.ds(WROW * t, WROW)],
                        cnt.at[pl.ds(pl.multiple_of(lo + WROW * t, 8),
                                     WROW)])
        plsc.subcore_barrier()
        return 0

    lax.fori_loop(0, NR // 2, one_pass, 0)


def _counts(dstR):
    return pl.kernel(
        _cnt_body,
        out_type=jax.ShapeDtypeStruct((NP, H), jnp.float32),
        mesh=_SC_MESH,
        scratch_types=[
            pltpu.VMEM((128, H), jnp.float32),
            pltpu.VMEM((128, H), jnp.float32),
            pltpu.VMEM((8, 128), jnp.int32),
            pltpu.VMEM((272,), jnp.int32),
            pltpu.VMEM((1, 128), jnp.int32),
            pltpu.VMEM((16,), jnp.int32),
            pltpu.VMEM_SHARED((ACCR, H), jnp.float32),
        ],
    )(dstR)


def kernel(node_features, edge_index, edge_features, constraint_types,
           enc_W1, enc_b1, enc_g, enc_be, enc_W2, enc_b2,
           msg_W1, msg_b1, msg_W2, msg_b2, upd_W, upd_b, ln_g, ln_b,
           att_W1, att_b1, att_W2, att_b2,
           out_W1, out_b1, out_W2, out_b2):
    src = edge_index[0]
    dst = edge_index[1]
    pad = jnp.full((NEP - NE,), DUMMY, jnp.int32)
    srcR = jnp.concatenate([src, pad]).reshape(NEP // 128, 128)
    dstR = jnp.concatenate([dst, pad]).reshape(NEP // 128, 128)

    h = _encoder(node_features, enc_W1, enc_b1, enc_g, enc_be, enc_W2, enc_b2)
    cnt = _counts(dstR)

    for i in range(L):
        w1 = msg_W1[i]
        w_ab = jnp.concatenate([w1[:H], w1[H:2 * H]], axis=1)
        bias_ab = jnp.concatenate(
            [msg_b1[i], jnp.zeros_like(msg_b1[i])]).reshape(1, 2 * H)
        a, b = _ab(h, w_ab, bias_ab)
        e = _eproj(edge_features, w1[2 * H:])
        p = _sc_edges(a, b, e, srcR, dstR)
        h = _update(h, p, cnt, msg_W2[i], msg_b2[i],
                    upd_W[i][:H], upd_W[i][H:], upd_b[i], ln_g[i], ln_b[i])

    out = _pool_head(h, att_W1, att_b1, att_W2, att_b2,
                     out_W1, out_b1, out_W2, out_b2)
    return out.reshape(D_OUT)


# speedup vs baseline: 3.4008x; 1.6406x over previous
import jax
import jax.numpy as jnp
from jax import lax
from jax.experimental import pallas as pl
from jax.experimental.pallas import tpu as pltpu
from jax.experimental.pallas import tpu_sc as plsc

N = 50000
NE = 800000
D_IN = 64
D_E = 16
H = 128
L = 4
D_OUT = 32

NBLK = 1000
EBLK = 8000

NT = 16
NR = 6
NQ = 8448
NP = NR * NQ
NEP = 819200
EPT = NEP // NT
NJR = EPT // 128
ACCR = NQ + 8
ZROW = NQ // NT
WROW = NQ // NT
DUMMY = N


def _ln(x, g, b, eps=1e-5):
    m = x.mean(-1, keepdims=True)
    v = ((x - m) ** 2).mean(-1, keepdims=True)
    return (x - m) * lax.rsqrt(v + eps) * g + b


def _enc_body(x_ref, w1_ref, b1_ref, g_ref, be_ref, w2_ref, b2_ref, o_ref):
    x = x_ref[...]
    h = jnp.maximum(x @ w1_ref[...] + b1_ref[...], 0.0)
    h = _ln(h, g_ref[...], be_ref[...])
    o_ref[...] = jnp.maximum(h @ w2_ref[...] + b2_ref[...], 0.0)


def _encoder(x, w1, b1, g, be, w2, b2):
    grid = (N // NBLK,)
    full = lambda r, c: pl.BlockSpec((r, c), lambda i: (0, 0))
    return pl.pallas_call(
        _enc_body,
        grid=grid,
        in_specs=[
            pl.BlockSpec((NBLK, D_IN), lambda i: (i, 0)),
            full(D_IN, H), full(1, H), full(1, H), full(1, H),
            full(H, H), full(1, H),
        ],
        out_specs=pl.BlockSpec((NBLK, H), lambda i: (i, 0)),
        out_shape=jax.ShapeDtypeStruct((N, H), jnp.float32),
    )(x, w1, b1.reshape(1, H), g.reshape(1, H), be.reshape(1, H),
      w2, b2.reshape(1, H))


def _ab_body(h_ref, w_ref, b_ref, a_ref, bo_ref):
    h = h_ref[...]
    ab = h @ w_ref[...] + b_ref[...]
    a_ref[...] = ab[:, :H]
    bo_ref[...] = ab[:, H:]


def _ab(h, w_ab, bias_ab):
    grid = (N // NBLK,)
    return pl.pallas_call(
        _ab_body,
        grid=grid,
        in_specs=[
            pl.BlockSpec((NBLK, H), lambda i: (i, 0)),
            pl.BlockSpec((H, 2 * H), lambda i: (0, 0)),
            pl.BlockSpec((1, 2 * H), lambda i: (0, 0)),
        ],
        out_specs=[
            pl.BlockSpec((NBLK, H), lambda i: (i, 0)),
            pl.BlockSpec((NBLK, H), lambda i: (i, 0)),
        ],
        out_shape=[
            jax.ShapeDtypeStruct((NP, H), jnp.float32),
            jax.ShapeDtypeStruct((NP, H), jnp.float32),
        ],
    )(h, w_ab, bias_ab)


def _e_body(e_ref, w_ref, o_ref):
    o_ref[...] = e_ref[...] @ w_ref[...]


def _eproj(ef, w_e):
    grid = (NE // EBLK,)
    return pl.pallas_call(
        _e_body,
        grid=grid,
        in_specs=[
            pl.BlockSpec((EBLK, D_E), lambda i: (i, 0)),
            pl.BlockSpec((D_E, H), lambda i: (0, 0)),
        ],
        out_specs=pl.BlockSpec((EBLK, H), lambda i: (i, 0)),
        out_shape=jax.ShapeDtypeStruct((NEP, H), jnp.float32),
    )(ef, w_e)


def _upd_body(h_ref, p_ref, cnt_ref, w2_ref, b2_ref, wuh_ref, wua_ref,
              ub_ref, g_ref, b_ref, o_ref):
    cnt = cnt_ref[...][:, 0:1]
    agg = (p_ref[...] @ w2_ref[...] + cnt * b2_ref[...]) / (cnt + 1e-8)
    h = h_ref[...]
    hn = jnp.maximum(h @ wuh_ref[...] + agg @ wua_ref[...] + ub_ref[...], 0.0)
    o_ref[...] = _ln(h + hn, g_ref[...], b_ref[...])


def _update(h, p, cnt, w2, b2, wuh, wua, ub, g, b):
    grid = (N // NBLK,)
    full = lambda r, c: pl.BlockSpec((r, c), lambda i: (0, 0))
    return pl.pallas_call(
        _upd_body,
        grid=grid,
        in_specs=[
            pl.BlockSpec((NBLK, H), lambda i: (i, 0)),
            pl.BlockSpec((NBLK, H), lambda i: (i, 0)),
            pl.BlockSpec((NBLK, H), lambda i: (i, 0)),
            full(H, H), full(1, H), full(H, H), full(H, H),
            full(1, H), full(1, H), full(1, H),
        ],
        out_specs=pl.BlockSpec((NBLK, H), lambda i: (i, 0)),
        out_shape=jax.ShapeDtypeStruct((N, H), jnp.float32),
    )(h, p, cnt, w2, b2.reshape(1, H), wuh, wua,
      ub.reshape(1, H), g.reshape(1, H), b.reshape(1, H))


def _pool_body(h_ref, aw1_ref, ab1_ref, aw2_ref, ab2_ref,
               ow1_ref, ob1_ref, ow2_ref, ob2_ref, o_ref,
               sexp_ref, gacc_ref):
    i = pl.program_id(0)

    @pl.when(i == 0)
    def _():
        sexp_ref[0, 0] = 0.0
        gacc_ref[...] = jnp.zeros_like(gacc_ref)

    h = h_ref[...]
    t = jnp.tanh(h @ aw1_ref[...] + ab1_ref[...])
    att = t @ aw2_ref[...] + ab2_ref[...]
    w = jnp.exp(att)
    sexp_ref[0, 0] += jnp.sum(w)
    gacc_ref[...] += (w * h).sum(axis=0, keepdims=True)

    @pl.when(i == pl.num_programs(0) - 1)
    def _():
        g = gacc_ref[...] / sexp_ref[0, 0]
        o_ref[...] = jnp.maximum(g @ ow1_ref[...] + ob1_ref[...], 0.0) \
            @ ow2_ref[...] + ob2_ref[...]


def _pool_head(h, aw1, ab1, aw2, ab2, ow1, ob1, ow2, ob2):
    grid = (N // NBLK,)
    full = lambda r, c: pl.BlockSpec((r, c), lambda i: (0, 0))
    return pl.pallas_call(
        _pool_body,
        grid=grid,
        in_specs=[
            pl.BlockSpec((NBLK, H), lambda i: (i, 0)),
            full(H, H // 2), full(1, H // 2), full(H // 2, 1), full(1, 1),
            full(H, H), full(1, H), full(H, D_OUT), full(1, D_OUT),
        ],
        out_specs=pl.BlockSpec((1, D_OUT), lambda i: (0, 0)),
        out_shape=jax.ShapeDtypeStruct((1, D_OUT), jnp.float32),
        scratch_shapes=[
            pltpu.SMEM((1, 1), jnp.float32),
            pltpu.VMEM((1, H), jnp.float32),
        ],
    )(h, aw1, ab1.reshape(1, H // 2), aw2, ab2.reshape(1, 1),
      ow1, ob1.reshape(1, H), ow2, ob2.reshape(1, D_OUT))


_SC_MESH = plsc.VectorSubcoreMesh(core_axis_name="c", subcore_axis_name="s")


def _dyn_gather(x, idx):
    dnums = lax.GatherDimensionNumbers(
        offset_dims=(), collapsed_slice_dims=(0,), start_index_map=(0,))
    return lax.gather(x, idx[:, None], dnums, (1,),
                      mode=lax.GatherScatterMode.PROMISE_IN_BOUNDS)


def _prefix16(mi, lane):
    cs = mi
    for sh in (1, 2, 4, 8):
        g = _dyn_gather(cs, jnp.maximum(lane - sh, 0))
        cs = cs + jnp.where(lane >= sh, g, 0)
    return cs


def _compact_perm(cs, lane):
    lo = jnp.zeros((16,), jnp.int32)
    tgt = lane + 1
    for step in (8, 4, 2, 1):
        probe = jnp.minimum(lo + (step - 1), 15)
        val = _dyn_gather(cs, probe)
        lo = jnp.where(val < tgt, lo + step, lo)
    return jnp.minimum(lo, 15)


def _vzero(buf, rows):
    def body(k, _):
        for v in range(8):
            buf[k, pl.ds(16 * v, 16)] = jnp.zeros((16,), jnp.float32)
        return 0
    lax.fori_loop(0, rows, body, 0)


def _zero_acc_rows(zbuf, acc, t):
    for z in range(ZROW // 128):
        pltpu.sync_copy(zbuf, acc.at[pl.ds(ZROW * t + 128 * z, 128)])
    rem = ZROW % 128
    if rem:
        pltpu.sync_copy(zbuf.at[pl.ds(0, rem)],
                        acc.at[pl.ds(ZROW * t + ZROW - rem, rem)])


def _stage_full(fsrc, fdstl, feid, i2s, i2dl, i2dg, i2e, lo):
    for v in range(8):
        sl = pl.ds(16 * v, 16)
        i2s[0, sl] = fsrc[sl]
        dl = fdstl[sl]
        i2dl[0, sl] = dl
        i2dg[0, sl] = dl + lo
        i2e[0, sl] = feid[sl]
    for v in range(8):
        fsrc[pl.ds(16 * v, 16)] = fsrc[pl.ds(128 + 16 * v, 16)]
        fdstl[pl.ds(16 * v, 16)] = fdstl[pl.ds(128 + 16 * v, 16)]
        feid[pl.ds(16 * v, 16)] = feid[pl.ds(128 + 16 * v, 16)]


def _stage_tail(fsrc, fdstl, feid, i2s, i2dl, i2dg, i2e, lo, cur):
    for v in range(8):
        sl = pl.ds(16 * v, 16)
        w = lax.iota(jnp.int32, 16) + (16 * v) < cur
        i2s[0, sl] = jnp.where(w, fsrc[sl], DUMMY)
        dl = fdstl[sl]
        i2dl[0, sl] = jnp.where(w, dl, NQ)
        i2dg[0, sl] = jnp.where(w, dl + lo, DUMMY)
        i2e[0, sl] = jnp.where(w, feid[sl], NE)


def _fire_gathers(at, bt, et, i2s, i2dg, i2e, a_buf, b_buf, e_buf, sem):
    pltpu.async_copy(at.at[i2s.at[0]], a_buf, sem)
    pltpu.async_copy(bt.at[i2dg.at[0]], b_buf, sem)
    pltpu.async_copy(et.at[i2e.at[0]], e_buf, sem)


def _finish_batch(et, i2dl, a_buf, b_buf, e_buf, acc, sem):
    pltpu.make_async_copy(et.at[pl.ds(0, 128)], a_buf, sem).wait()
    pltpu.make_async_copy(et.at[pl.ds(0, 128)], b_buf, sem).wait()
    pltpu.make_async_copy(et.at[pl.ds(0, 128)], e_buf, sem).wait()

    def rbody(k, _):
        for v in range(8):
            sl = pl.ds(16 * v, 16)
            s = a_buf[k, sl] + b_buf[k, sl] + e_buf[k, sl]
            a_buf[k, sl] = jnp.maximum(s, 0.0)
        return 0
    lax.fori_loop(0, 128, rbody, 0)

    pltpu.sync_copy(a_buf, acc.at[i2dl.at[0]], add=True)


def _edge_body(at, bt, et, srcR, dstR, p_out,
               a_buf, b_buf, e_buf, sbuf, dbuf, fsrc, fdstl, feid,
               i2s, i2dl, i2dg, i2e, xbuf, acc, sem, semi):
    c = lax.axis_index("c")
    t = lax.axis_index("s")

    def one_pass(p, _):
        q = (NR // 2) * c + p
        lo = q * NQ

        _vzero(e_buf, 128)
        _zero_acc_rows(e_buf, acc, t)
        plsc.subcore_barrier()

        row0 = pl.multiple_of(t * NJR, 8)
        pltpu.async_copy(srcR.at[pl.ds(row0, 8)], sbuf.at[pl.ds(0, 8)], semi)
        pltpu.async_copy(dstR.at[pl.ds(row0, 8)], dbuf.at[pl.ds(0, 8)], semi)

        def jrow(j, carry):
            cur, pending = carry

            @pl.when(j % 8 == 0)
            def _():
                half = (j // 8) % 2
                pltpu.make_async_copy(
                    srcR.at[pl.ds(0, 8)], sbuf.at[pl.ds(half * 8, 8)],
                    semi).wait()
                pltpu.make_async_copy(
                    dstR.at[pl.ds(0, 8)], dbuf.at[pl.ds(half * 8, 8)],
                    semi).wait()

                @pl.when(j + 8 < NJR)
                def _():
                    r2 = pl.multiple_of(t * NJR + j + 8, 8)
                    oth = 8 - half * 8
                    pltpu.async_copy(srcR.at[pl.ds(r2, 8)],
                                     sbuf.at[pl.ds(oth, 8)], semi)
                    pltpu.async_copy(dstR.at[pl.ds(r2, 8)],
                                     dbuf.at[pl.ds(oth, 8)], semi)

            jj = j % 16
            base = t * EPT + j * 128
            lane = lax.iota(jnp.int32, 16)
            for v in range(8):
                sl = pl.ds(16 * v, 16)
                d = dbuf[jj, sl]
                m = (d >= lo) & (d < lo + NQ)
                cs = _prefix16(jnp.where(m, 1, 0), lane)
                xbuf[pl.ds(0, 16)] = cs
                perm = _compact_perm(cs, lane)
                fsrc[pl.ds(cur, 16)] = _dyn_gather(sbuf[jj, sl], perm)
                fdstl[pl.ds(cur, 16)] = _dyn_gather(d - lo, perm)
                fe = perm + (base + 16 * v)
                feid[pl.ds(cur, 16)] = fe
                cur = cur + xbuf[pl.ds(0, 16)][15]

            full = cur >= 128

            @pl.when(full & (pending > 0))
            def _():
                _finish_batch(et, i2dl, a_buf, b_buf, e_buf, acc, sem)

            @pl.when(full)
            def _():
                _stage_full(fsrc, fdstl, feid, i2s, i2dl, i2dg, i2e, lo)
                _fire_gathers(at, bt, et, i2s, i2dg, i2e,
                              a_buf, b_buf, e_buf, sem)
            return (jnp.where(full, cur - 128, cur),
                    jnp.where(full, 1, pending))

        cur, pending = lax.fori_loop(
            0, NJR, jrow, (jnp.int32(0), jnp.int32(0)))

        @pl.when(pending > 0)
        def _():
            _finish_batch(et, i2dl, a_buf, b_buf, e_buf, acc, sem)

        @pl.when(cur > 0)
        def _():
            _stage_tail(fsrc, fdstl, feid, i2s, i2dl, i2dg, i2e, lo, cur)
            _fire_gathers(at, bt, et, i2s, i2dg, i2e,
                          a_buf, b_buf, e_buf, sem)
            _finish_batch(et, i2dl, a_buf, b_buf, e_buf, acc, sem)

        plsc.subcore_barrier()
        pltpu.sync_copy(acc.at[pl.ds(WROW * t, WROW)],
                        p_out.at[pl.ds(pl.multiple_of(lo + WROW * t, 8),
                                       WROW)])
        plsc.subcore_barrier()
        return 0

    lax.fori_loop(0, NR // 2, one_pass, 0)


def _sc_edges(at, bt, et, srcR, dstR):
    return pl.kernel(
        _edge_body,
        out_type=jax.ShapeDtypeStruct((NP, H), jnp.float32),
        mesh=_SC_MESH,
        scratch_types=[
            pltpu.VMEM((128, H), jnp.float32),
            pltpu.VMEM((128, H), jnp.float32),
            pltpu.VMEM((128, H), jnp.float32),
            pltpu.VMEM((16, 128), jnp.int32),
            pltpu.VMEM((16, 128), jnp.int32),
            pltpu.VMEM((272,), jnp.int32),
            pltpu.VMEM((272,), jnp.int32),
            pltpu.VMEM((272,), jnp.int32),
            pltpu.VMEM((1, 128), jnp.int32),
            pltpu.VMEM((1, 128), jnp.int32),
            pltpu.VMEM((1, 128), jnp.int32),
            pltpu.VMEM((1, 128), jnp.int32),
            pltpu.VMEM((16,), jnp.int32),
            pltpu.VMEM_SHARED((ACCR, H), jnp.float32),
            pltpu.SemaphoreType.DMA,
            pltpu.SemaphoreType.DMA,
        ],
    )(at, bt, et, srcR, dstR)


def _cnt_body(dstR, cnt, ones, zbuf, dbuf, fdstl, i2dl, xbuf, acc):
    c = lax.axis_index("c")
    t = lax.axis_index("s")

    def ofill(k, _):
        for v in range(8):
            ones[k, pl.ds(16 * v, 16)] = jnp.ones((16,), jnp.float32)
        return 0
    lax.fori_loop(0, 128, ofill, 0)
    _vzero(zbuf, 128)

    def one_pass(p, _):
        q = (NR // 2) * c + p
        lo = q * NQ

        _zero_acc_rows(zbuf, acc, t)
        plsc.subcore_barrier()

        def jrow(j, cur):
            @pl.when(j % 8 == 0)
            def _():
                pltpu.sync_copy(
                    dstR.at[pl.ds(pl.multiple_of(t * NJR + j, 8), 8)], dbuf)
            jj = j % 8
            lane = lax.iota(jnp.int32, 16)
            for v in range(8):
                d = dbuf[jj, pl.ds(16 * v, 16)]
                m = (d >= lo) & (d < lo + NQ)
                cs = _prefix16(jnp.where(m, 1, 0), lane)
                xbuf[pl.ds(0, 16)] = cs
                perm = _compact_perm(cs, lane)
                fdstl[pl.ds(cur, 16)] = _dyn_gather(d - lo, perm)
                cur = cur + xbuf[pl.ds(0, 16)][15]

            full = cur >= 128

            @pl.when(full)
            def _():
                for v in range(8):
                    i2dl[0, pl.ds(16 * v, 16)] = fdstl[pl.ds(16 * v, 16)]
                for v in range(8):
                    fdstl[pl.ds(16 * v, 16)] = fdstl[pl.ds(128 + 16 * v, 16)]
                pltpu.sync_copy(ones, acc.at[i2dl.at[0]], add=True)
            return jnp.where(full, cur - 128, cur)

        cur = lax.fori_loop(0, NJR, jrow, jnp.int32(0))

        @pl.when(cur > 0)
        def _():
            for v in range(8):
                sl = pl.ds(16 * v, 16)
                w = lax.iota(jnp.int32, 16) + (16 * v) < cur
                i2dl[0, sl] = jnp.where(w, fdstl[sl], NQ)
            pltpu.sync_copy(ones, acc.at[i2dl.at[0]], add=True)

        plsc.subcore_barrier()
        pltpu.sync_copy(acc.at[pl.ds(WROW * t, WROW)],
                        cnt.at[pl.ds(pl.multiple_of(lo + WROW * t, 8),
                                     WROW)])
        plsc.subcore_barrier()
        return 0

    lax.fori_loop(0, NR // 2, one_pass, 0)


def _counts(dstR):
    return pl.kernel(
        _cnt_body,
        out_type=jax.ShapeDtypeStruct((NP, H), jnp.float32),
        mesh=_SC_MESH,
        scratch_types=[
            pltpu.VMEM((128, H), jnp.float32),
            pltpu.VMEM((128, H), jnp.float32),
            pltpu.VMEM((8, 128), jnp.int32),
            pltpu.VMEM((272,), jnp.int32),
            pltpu.VMEM((1, 128), jnp.int32),
            pltpu.VMEM((16,), jnp.int32),
            pltpu.VMEM_SHARED((ACCR, H), jnp.float32),
        ],
    )(dstR)


def kernel(node_features, edge_index, edge_features, constraint_types,
           enc_W1, enc_b1, enc_g, enc_be, enc_W2, enc_b2,
           msg_W1, msg_b1, msg_W2, msg_b2, upd_W, upd_b, ln_g, ln_b,
           att_W1, att_b1, att_W2, att_b2,
           out_W1, out_b1, out_W2, out_b2):
    src = edge_index[0]
    dst = edge_index[1]
    pad = jnp.full((NEP - NE,), DUMMY, jnp.int32)
    srcR = jnp.concatenate([src, pad]).reshape(NEP // 128, 128)
    dstR = jnp.concatenate([dst, pad]).reshape(NEP // 128, 128)

    h = _encoder(node_features, enc_W1, enc_b1, enc_g, enc_be, enc_W2, enc_b2)
    cnt = _counts(dstR)

    for i in range(L):
        w1 = msg_W1[i]
        w_ab = jnp.concatenate([w1[:H], w1[H:2 * H]], axis=1)
        bias_ab = jnp.concatenate(
            [msg_b1[i], jnp.zeros_like(msg_b1[i])]).reshape(1, 2 * H)
        a, b = _ab(h, w_ab, bias_ab)
        e = _eproj(edge_features, w1[2 * H:])
        p = _sc_edges(a, b, e, srcR, dstR)
        h = _update(h, p, cnt, msg_W2[i], msg_b2[i],
                    upd_W[i][:H], upd_W[i][H:], upd_b[i], ln_g[i], ln_b[i])

    out = _pool_head(h, att_W1, att_b1, att_W2, att_b2,
                     out_W1, out_b1, out_W2, out_b2)
    return out.reshape(D_OUT)
